# Initial kernel scaffold; baseline (speedup 1.0000x reference)
import jax
import jax.numpy as jnp
from jax.experimental import pallas as pl


def kernel(uid, iid, mu, upsilon, uid_alpha_emb, iid_alpha_emb, uid_beta_emb, iid_beta_emb, g_alpha_bias, g_beta_bias):
    raise NotImplementedError("write your pallas kernel here")


# speedup vs baseline: 1.0987x; 1.0987x over previous
import functools

import jax
import jax.numpy as jnp
from jax import lax
from jax.experimental import pallas as pl
from jax.experimental.pallas import tpu as pltpu
from jax.experimental.pallas import tpu_sc as plsc

_B = 16384
_L = 16
_NC = 2
_NS = 16
_NW = _NC * _NS
_BW = _B // _NW


def _sc_body(uid_hbm, iid_hbm, mu_hbm, up_hbm,
             ua_hbm, ia_hbm, ub_hbm, ib_hbm, gab_hbm, gbb_hbm,
             alpha_hbm, beta_hbm,
             uidx_v, iidx_v, mu_v, up_v, ua_v, ia_v, ub_v, ib_v,
             a_v, b_v, gab_v, gbb_v, sem):
    wid = lax.axis_index("s") * _NC + lax.axis_index("c")
    base = wid * _BW

    pltpu.sync_copy(uid_hbm.at[pl.ds(base, _BW)], uidx_v)
    pltpu.sync_copy(iid_hbm.at[pl.ds(base, _BW)], iidx_v)
    c1 = pltpu.async_copy(ua_hbm.at[uidx_v], ua_v, sem)
    c2 = pltpu.async_copy(ia_hbm.at[iidx_v], ia_v, sem)
    c3 = pltpu.async_copy(ub_hbm.at[uidx_v], ub_v, sem)
    c4 = pltpu.async_copy(ib_hbm.at[iidx_v], ib_v, sem)
    pltpu.sync_copy(mu_hbm.at[pl.ds(base, _BW)], mu_v)
    pltpu.sync_copy(up_hbm.at[pl.ds(base, _BW)], up_v)
    pltpu.sync_copy(gab_hbm, gab_v)
    pltpu.sync_copy(gbb_hbm, gbb_v)
    c1.wait()
    c2.wait()
    c3.wait()
    c4.wait()

    eps = jnp.float32(0.01)
    ga = gab_v[...]
    gb = gbb_v[...]
    for i in range(_BW // _L):
        sl = pl.ds(i * _L, _L)
        mu16 = mu_v[sl]
        up16 = up_v[sl]
        al = jnp.maximum(mu16 * up16, eps) + ga + ua_v[sl] + ia_v[sl]
        be = jnp.maximum(up16 - jnp.maximum(mu16 * up16, eps), eps) + gb
        a_v[sl] = jnp.maximum(al, eps)
        b_v[sl] = jnp.maximum(be + ub_v[sl] + ib_v[sl], eps)

    pltpu.sync_copy(a_v, alpha_hbm.at[pl.ds(base, _BW)])
    pltpu.sync_copy(b_v, beta_hbm.at[pl.ds(base, _BW)])


_sc_call = pl.kernel(
    _sc_body,
    out_type=(jax.ShapeDtypeStruct((_B,), jnp.float32),
              jax.ShapeDtypeStruct((_B,), jnp.float32)),
    mesh=plsc.VectorSubcoreMesh(core_axis_name="c", subcore_axis_name="s"),
    scratch_types=[
        pltpu.VMEM((_BW,), jnp.int32),
        pltpu.VMEM((_BW,), jnp.int32),
        pltpu.VMEM((_BW,), jnp.float32),
        pltpu.VMEM((_BW,), jnp.float32),
        pltpu.VMEM((_BW,), jnp.float32),
        pltpu.VMEM((_BW,), jnp.float32),
        pltpu.VMEM((_BW,), jnp.float32),
        pltpu.VMEM((_BW,), jnp.float32),
        pltpu.VMEM((_BW,), jnp.float32),
        pltpu.VMEM((_BW,), jnp.float32),
        pltpu.VMEM((_L,), jnp.float32),
        pltpu.VMEM((_L,), jnp.float32),
        pltpu.SemaphoreType.DMA,
    ],
)


@jax.jit
def kernel(uid, iid, mu, upsilon, uid_alpha_emb, iid_alpha_emb,
           uid_beta_emb, iid_beta_emb, g_alpha_bias, g_beta_bias):
    ga = jnp.full((_L,), g_alpha_bias, jnp.float32)
    gb = jnp.full((_L,), g_beta_bias, jnp.float32)
    alpha, beta = _sc_call(
        uid.astype(jnp.int32), iid.astype(jnp.int32),
        mu.reshape(-1), upsilon.reshape(-1),
        uid_alpha_emb.reshape(-1), iid_alpha_emb.reshape(-1),
        uid_beta_emb.reshape(-1), iid_beta_emb.reshape(-1),
        ga, gb)
    return (mu, upsilon, alpha.reshape(-1, 1), beta.reshape(-1, 1))


# speedup vs baseline: 4.0296x; 3.6677x over previous
import functools

import jax
import jax.numpy as jnp
from jax import lax
from jax.experimental import pallas as pl
from jax.experimental.pallas import tpu as pltpu
from jax.experimental.pallas import tpu_sc as plsc

_B = 16384
_L = 16
_NC = 2
_NS = 16
_NW = _NC * _NS
_BW = _B // _NW
_N = 1000000
_NPAD = 1000448


def _sc_body(uid_hbm, iid_hbm, mu_hbm, up_hbm,
             ua_hbm, ia_hbm, ub_hbm, ib_hbm, gab_hbm, gbb_hbm,
             alpha_hbm, beta_hbm,
             uidx_v, iidx_v, mu_v, up_v, ua_v, ia_v, ub_v, ib_v,
             a_v, b_v, gab_v, gbb_v, sem):
    wid = lax.axis_index("s") * _NC + lax.axis_index("c")
    base = wid * _BW

    pltpu.sync_copy(uid_hbm.at[pl.ds(base, _BW)], uidx_v)
    pltpu.sync_copy(iid_hbm.at[pl.ds(base, _BW)], iidx_v)
    c1 = pltpu.async_copy(ua_hbm.at[uidx_v], ua_v, sem)
    c2 = pltpu.async_copy(ia_hbm.at[iidx_v], ia_v, sem)
    c3 = pltpu.async_copy(ub_hbm.at[uidx_v], ub_v, sem)
    c4 = pltpu.async_copy(ib_hbm.at[iidx_v], ib_v, sem)
    pltpu.sync_copy(mu_hbm.at[pl.ds(base, _BW)], mu_v)
    pltpu.sync_copy(up_hbm.at[pl.ds(base, _BW)], up_v)
    pltpu.sync_copy(gab_hbm, gab_v)
    pltpu.sync_copy(gbb_hbm, gbb_v)
    c1.wait()
    c2.wait()
    c3.wait()
    c4.wait()

    eps = jnp.float32(0.01)
    ga = gab_v[...]
    gb = gbb_v[...]
    for i in range(_BW // _L):
        sl = pl.ds(i * _L, _L)
        mu16 = mu_v[sl]
        up16 = up_v[sl]
        al = jnp.maximum(mu16 * up16, eps) + ga + ua_v[sl] + ia_v[sl]
        be = jnp.maximum(up16 - jnp.maximum(mu16 * up16, eps), eps) + gb
        a_v[sl] = jnp.maximum(al, eps)
        b_v[sl] = jnp.maximum(be + ub_v[sl] + ib_v[sl], eps)

    pltpu.sync_copy(a_v, alpha_hbm.at[pl.ds(base, _BW)])
    pltpu.sync_copy(b_v, beta_hbm.at[pl.ds(base, _BW)])


_sc_call = pl.kernel(
    _sc_body,
    out_type=(jax.ShapeDtypeStruct((_B,), jnp.float32),
              jax.ShapeDtypeStruct((_B,), jnp.float32)),
    mesh=plsc.VectorSubcoreMesh(core_axis_name="c", subcore_axis_name="s"),
    scratch_types=[
        pltpu.VMEM((_BW,), jnp.int32),
        pltpu.VMEM((_BW,), jnp.int32),
        pltpu.VMEM((_BW,), jnp.float32),
        pltpu.VMEM((_BW,), jnp.float32),
        pltpu.VMEM((_BW,), jnp.float32),
        pltpu.VMEM((_BW,), jnp.float32),
        pltpu.VMEM((_BW,), jnp.float32),
        pltpu.VMEM((_BW,), jnp.float32),
        pltpu.VMEM((_BW,), jnp.float32),
        pltpu.VMEM((_BW,), jnp.float32),
        pltpu.VMEM((_L,), jnp.float32),
        pltpu.VMEM((_L,), jnp.float32),
        pltpu.SemaphoreType.DMA,
    ],
)


def _flat_table(t):
    return jnp.pad(t, ((0, _NPAD - _N), (0, 0))).reshape(-1)


@jax.jit
def kernel(uid, iid, mu, upsilon, uid_alpha_emb, iid_alpha_emb,
           uid_beta_emb, iid_beta_emb, g_alpha_bias, g_beta_bias):
    ga = jnp.full((_L,), g_alpha_bias, jnp.float32)
    gb = jnp.full((_L,), g_beta_bias, jnp.float32)
    alpha, beta = _sc_call(
        uid.astype(jnp.int32), iid.astype(jnp.int32),
        mu.reshape(-1), upsilon.reshape(-1),
        _flat_table(uid_alpha_emb), _flat_table(iid_alpha_emb),
        _flat_table(uid_beta_emb), _flat_table(iid_beta_emb),
        ga, gb)
    return (mu, upsilon, alpha.reshape(-1, 1), beta.reshape(-1, 1))


# speedup vs baseline: 4.0364x; 1.0017x over previous
import functools

import jax
import jax.numpy as jnp
from jax import lax
from jax.experimental import pallas as pl
from jax.experimental.pallas import tpu as pltpu
from jax.experimental.pallas import tpu_sc as plsc

_B = 16384
_L = 16
_NC = 2
_NS = 16
_NW = _NC * _NS
_BW = _B // _NW
_N = 1000000
_NPAD = 1000448


def _sc_body(uid_hbm, iid_hbm, mu_hbm, up_hbm,
             ua_hbm, ia_hbm, ub_hbm, ib_hbm, gab_hbm, gbb_hbm,
             alpha_hbm, beta_hbm,
             uidx_v, iidx_v, mu_v, up_v, ua_v, ia_v, ub_v, ib_v,
             a_v, b_v, gab_v, gbb_v, sem):
    wid = lax.axis_index("s") * _NC + lax.axis_index("c")
    base = wid * _BW

    pltpu.sync_copy(uid_hbm.at[pl.ds(base, _BW)], uidx_v)
    pltpu.sync_copy(iid_hbm.at[pl.ds(base, _BW)], iidx_v)
    c1 = pltpu.async_copy(ua_hbm.at[uidx_v], ua_v, sem)
    c2 = pltpu.async_copy(ia_hbm.at[iidx_v], ia_v, sem)
    c3 = pltpu.async_copy(ub_hbm.at[uidx_v], ub_v, sem)
    c4 = pltpu.async_copy(ib_hbm.at[iidx_v], ib_v, sem)
    pltpu.sync_copy(mu_hbm.at[pl.ds(base, _BW)], mu_v)
    pltpu.sync_copy(up_hbm.at[pl.ds(base, _BW)], up_v)
    pltpu.sync_copy(gab_hbm, gab_v)
    pltpu.sync_copy(gbb_hbm, gbb_v)
    c1.wait()
    c2.wait()
    c3.wait()
    c4.wait()

    eps = jnp.float32(0.01)
    ga = gab_v[...]
    gb = gbb_v[...]
    for i in range(_BW // _L):
        sl = pl.ds(i * _L, _L)
        mu16 = mu_v[sl]
        up16 = up_v[sl]
        al = jnp.maximum(mu16 * up16, eps) + ga + ua_v[sl] + ia_v[sl]
        be = jnp.maximum(up16 - jnp.maximum(mu16 * up16, eps), eps) + gb
        a_v[sl] = jnp.maximum(al, eps)
        b_v[sl] = jnp.maximum(be + ub_v[sl] + ib_v[sl], eps)

    pltpu.sync_copy(a_v, alpha_hbm.at[pl.ds(base, _BW)])
    pltpu.sync_copy(b_v, beta_hbm.at[pl.ds(base, _BW)])


_sc_call = pl.kernel(
    _sc_body,
    out_type=(jax.ShapeDtypeStruct((_B,), jnp.float32),
              jax.ShapeDtypeStruct((_B,), jnp.float32)),
    mesh=plsc.VectorSubcoreMesh(core_axis_name="c", subcore_axis_name="s"),
    scratch_types=[
        pltpu.VMEM((_BW,), jnp.int32),
        pltpu.VMEM((_BW,), jnp.int32),
        pltpu.VMEM((_BW,), jnp.float32),
        pltpu.VMEM((_BW,), jnp.float32),
        pltpu.VMEM((_BW,), jnp.float32),
        pltpu.VMEM((_BW,), jnp.float32),
        pltpu.VMEM((_BW,), jnp.float32),
        pltpu.VMEM((_BW,), jnp.float32),
        pltpu.VMEM((_BW,), jnp.float32),
        pltpu.VMEM((_BW,), jnp.float32),
        pltpu.VMEM((_L,), jnp.float32),
        pltpu.VMEM((_L,), jnp.float32),
        pltpu.SemaphoreType.DMA,
    ],
)


def _flat_table(t):
    zpad = jnp.zeros((_NPAD - _N, 1), jnp.float32)
    return jnp.concatenate([t, zpad], axis=0).reshape(-1)


@jax.jit
def kernel(uid, iid, mu, upsilon, uid_alpha_emb, iid_alpha_emb,
           uid_beta_emb, iid_beta_emb, g_alpha_bias, g_beta_bias):
    ga = jnp.full((_L,), g_alpha_bias, jnp.float32)
    gb = jnp.full((_L,), g_beta_bias, jnp.float32)
    alpha, beta = _sc_call(
        uid.astype(jnp.int32), iid.astype(jnp.int32),
        mu.reshape(-1), upsilon.reshape(-1),
        _flat_table(uid_alpha_emb), _flat_table(iid_alpha_emb),
        _flat_table(uid_beta_emb), _flat_table(iid_beta_emb),
        ga, gb)
    return (mu, upsilon, alpha.reshape(-1, 1), beta.reshape(-1, 1))


# speedup vs baseline: 4.0468x; 1.0026x over previous
import functools

import jax
import jax.numpy as jnp
from jax import lax
from jax.experimental import pallas as pl
from jax.experimental.pallas import tpu as pltpu
from jax.experimental.pallas import tpu_sc as plsc

_B = 16384
_L = 16
_NC = 2
_NS = 16
_NW = _NC * _NS
_BW = _B // _NW
_N = 1000000
_NPAD = 1000448


def _sc_body(uid_hbm, iid_hbm, mu_hbm, up_hbm,
             ua_hbm, ia_hbm, ub_hbm, ib_hbm, gab_hbm, gbb_hbm,
             mu_out, up_out, alpha_hbm, beta_hbm,
             uidx_v, iidx_v, mu_v, up_v, ua_v, ia_v, ub_v, ib_v,
             a_v, b_v, gab_v, gbb_v, sem, sem2):
    wid = lax.axis_index("s") * _NC + lax.axis_index("c")
    base = wid * _BW
    sl_w = pl.ds(base, _BW)

    pltpu.sync_copy(uid_hbm.at[sl_w], uidx_v)
    pltpu.sync_copy(iid_hbm.at[sl_w], iidx_v)
    c1 = pltpu.async_copy(ua_hbm.at[uidx_v], ua_v, sem)
    c2 = pltpu.async_copy(ia_hbm.at[iidx_v], ia_v, sem)
    c3 = pltpu.async_copy(ub_hbm.at[uidx_v], ub_v, sem)
    c4 = pltpu.async_copy(ib_hbm.at[iidx_v], ib_v, sem)
    m1 = pltpu.async_copy(mu_hbm.at[sl_w], mu_out.at[sl_w], sem2)
    m2 = pltpu.async_copy(up_hbm.at[sl_w], up_out.at[sl_w], sem2)
    pltpu.sync_copy(mu_hbm.at[sl_w], mu_v)
    pltpu.sync_copy(up_hbm.at[sl_w], up_v)
    pltpu.sync_copy(gab_hbm, gab_v)
    pltpu.sync_copy(gbb_hbm, gbb_v)

    eps = jnp.float32(0.01)
    ga = gab_v[...]
    gb = gbb_v[...]

    c1.wait()
    c2.wait()
    for i in range(_BW // _L):
        sl = pl.ds(i * _L, _L)
        al = jnp.maximum(mu_v[sl] * up_v[sl], eps) + ga + ua_v[sl] + ia_v[sl]
        a_v[sl] = jnp.maximum(al, eps)
    pltpu.sync_copy(a_v, alpha_hbm.at[sl_w])

    c3.wait()
    c4.wait()
    for i in range(_BW // _L):
        sl = pl.ds(i * _L, _L)
        up16 = up_v[sl]
        be = jnp.maximum(up16 - jnp.maximum(mu_v[sl] * up16, eps), eps) + gb
        b_v[sl] = jnp.maximum(be + ub_v[sl] + ib_v[sl], eps)
    pltpu.sync_copy(b_v, beta_hbm.at[sl_w])

    m1.wait()
    m2.wait()


_sc_call = pl.kernel(
    _sc_body,
    out_type=(jax.ShapeDtypeStruct((_B,), jnp.float32),
              jax.ShapeDtypeStruct((_B,), jnp.float32),
              jax.ShapeDtypeStruct((_B,), jnp.float32),
              jax.ShapeDtypeStruct((_B,), jnp.float32)),
    mesh=plsc.VectorSubcoreMesh(core_axis_name="c", subcore_axis_name="s"),
    scratch_types=[
        pltpu.VMEM((_BW,), jnp.int32),
        pltpu.VMEM((_BW,), jnp.int32),
        pltpu.VMEM((_BW,), jnp.float32),
        pltpu.VMEM((_BW,), jnp.float32),
        pltpu.VMEM((_BW,), jnp.float32),
        pltpu.VMEM((_BW,), jnp.float32),
        pltpu.VMEM((_BW,), jnp.float32),
        pltpu.VMEM((_BW,), jnp.float32),
        pltpu.VMEM((_BW,), jnp.float32),
        pltpu.VMEM((_BW,), jnp.float32),
        pltpu.VMEM((_L,), jnp.float32),
        pltpu.VMEM((_L,), jnp.float32),
        pltpu.SemaphoreType.DMA,
        pltpu.SemaphoreType.DMA,
    ],
)


def _flat_table(t):
    zpad = jnp.zeros((_NPAD - _N, 1), jnp.float32)
    return jnp.concatenate([t, zpad], axis=0).reshape(-1)


@jax.jit
def kernel(uid, iid, mu, upsilon, uid_alpha_emb, iid_alpha_emb,
           uid_beta_emb, iid_beta_emb, g_alpha_bias, g_beta_bias):
    ga = jnp.full((_L,), g_alpha_bias, jnp.float32)
    gb = jnp.full((_L,), g_beta_bias, jnp.float32)
    mu_o, up_o, alpha, beta = _sc_call(
        uid.astype(jnp.int32), iid.astype(jnp.int32),
        mu.reshape(-1), upsilon.reshape(-1),
        _flat_table(uid_alpha_emb), _flat_table(iid_alpha_emb),
        _flat_table(uid_beta_emb), _flat_table(iid_beta_emb),
        ga, gb)
    return (mu_o.reshape(-1, 1), up_o.reshape(-1, 1),
            alpha.reshape(-1, 1), beta.reshape(-1, 1))


# speedup vs baseline: 4.0528x; 1.0015x over previous
import functools

import jax
import jax.numpy as jnp
from jax import lax
from jax.experimental import pallas as pl
from jax.experimental.pallas import tpu as pltpu
from jax.experimental.pallas import tpu_sc as plsc

_B = 16384
_L = 16
_NC = 2
_NS = 16
_NW = _NC * _NS
_BW = _B // _NW
_N = 1000000
_NPAD = 1000448


def _sc_body(uid_hbm, iid_hbm, mu_hbm, up_hbm,
             ua_hbm, ia_hbm, ub_hbm, ib_hbm, gab_hbm, gbb_hbm,
             mu_out, up_out, alpha_hbm, beta_hbm,
             uidx_v, iidx_v, mu_v, up_v, ua_v, ia_v, ub_v, ib_v,
             a_v, b_v, gab_v, gbb_v, sem, sem2):
    wid = lax.axis_index("s") * _NC + lax.axis_index("c")
    base = wid * _BW
    sl_w = pl.ds(base, _BW)

    pltpu.sync_copy(uid_hbm.at[sl_w], uidx_v)
    pltpu.sync_copy(iid_hbm.at[sl_w], iidx_v)
    c1 = pltpu.async_copy(ua_hbm.at[uidx_v], ua_v, sem)
    c2 = pltpu.async_copy(ia_hbm.at[iidx_v], ia_v, sem)
    c3 = pltpu.async_copy(ub_hbm.at[uidx_v], ub_v, sem)
    c4 = pltpu.async_copy(ib_hbm.at[iidx_v], ib_v, sem)
    m1 = pltpu.async_copy(mu_hbm.at[sl_w], mu_out.at[sl_w], sem2)
    m2 = pltpu.async_copy(up_hbm.at[sl_w], up_out.at[sl_w], sem2)
    pltpu.sync_copy(mu_hbm.at[sl_w], mu_v)
    pltpu.sync_copy(up_hbm.at[sl_w], up_v)
    pltpu.sync_copy(gab_hbm, gab_v)
    pltpu.sync_copy(gbb_hbm, gbb_v)

    eps = jnp.float32(0.01)
    ga = gab_v[...]
    gb = gbb_v[...]

    c1.wait()
    c2.wait()
    for i in range(_BW // _L):
        sl = pl.ds(i * _L, _L)
        al = jnp.maximum(mu_v[sl] * up_v[sl], eps) + ga + ua_v[sl] + ia_v[sl]
        a_v[sl] = jnp.maximum(al, eps)
    pltpu.sync_copy(a_v, alpha_hbm.at[sl_w])

    c3.wait()
    c4.wait()
    for i in range(_BW // _L):
        sl = pl.ds(i * _L, _L)
        up16 = up_v[sl]
        be = jnp.maximum(up16 - jnp.maximum(mu_v[sl] * up16, eps), eps) + gb
        b_v[sl] = jnp.maximum(be + ub_v[sl] + ib_v[sl], eps)
    pltpu.sync_copy(b_v, beta_hbm.at[sl_w])

    m1.wait()
    m2.wait()


_sc_call = pl.kernel(
    _sc_body,
    out_type=(jax.ShapeDtypeStruct((_B,), jnp.float32),
              jax.ShapeDtypeStruct((_B,), jnp.float32),
              jax.ShapeDtypeStruct((_B,), jnp.float32),
              jax.ShapeDtypeStruct((_B,), jnp.float32)),
    mesh=plsc.VectorSubcoreMesh(core_axis_name="c", subcore_axis_name="s"),
    scratch_types=[
        pltpu.VMEM((_BW,), jnp.int32),
        pltpu.VMEM((_BW,), jnp.int32),
        pltpu.VMEM((_BW,), jnp.float32),
        pltpu.VMEM((_BW,), jnp.float32),
        pltpu.VMEM((_BW,), jnp.float32),
        pltpu.VMEM((_BW,), jnp.float32),
        pltpu.VMEM((_BW,), jnp.float32),
        pltpu.VMEM((_BW,), jnp.float32),
        pltpu.VMEM((_BW,), jnp.float32),
        pltpu.VMEM((_BW,), jnp.float32),
        pltpu.VMEM((_L,), jnp.float32),
        pltpu.VMEM((_L,), jnp.float32),
        pltpu.SemaphoreType.DMA,
        pltpu.SemaphoreType.DMA,
    ],
)


def _flat_table(t):
    buf = jnp.zeros((_NPAD, 1), jnp.float32)
    return lax.dynamic_update_slice(buf, t, (0, 0)).reshape(-1)


@jax.jit
def kernel(uid, iid, mu, upsilon, uid_alpha_emb, iid_alpha_emb,
           uid_beta_emb, iid_beta_emb, g_alpha_bias, g_beta_bias):
    ga = jnp.full((_L,), g_alpha_bias, jnp.float32)
    gb = jnp.full((_L,), g_beta_bias, jnp.float32)
    mu_o, up_o, alpha, beta = _sc_call(
        uid.astype(jnp.int32), iid.astype(jnp.int32),
        mu.reshape(-1), upsilon.reshape(-1),
        _flat_table(uid_alpha_emb), _flat_table(iid_alpha_emb),
        _flat_table(uid_beta_emb), _flat_table(iid_beta_emb),
        ga, gb)
    return (mu_o.reshape(-1, 1), up_o.reshape(-1, 1),
            alpha.reshape(-1, 1), beta.reshape(-1, 1))
